# baseline (speedup 1.0000x reference)
import functools

import jax
import jax.numpy as jnp
from jax import lax
from jax.experimental import pallas as pl
from jax.experimental.pallas import tpu as pltpu
from jax.experimental.pallas import tpu_sc as plsc

VOCAB = 100000
HIDDEN = 128
S = 200
B = 1024

NC = 2
NS = 16
NW = NC * NS
N_TOK = B * S
TOK_PER_W = N_TOK // NW
CHUNK = 80
N_CHUNKS = TOK_PER_W // CHUNK
NBUF = 4
N_OUTER = N_CHUNKS // NBUF
NSL = HIDDEN // 16


def _body(ids_hbm, tt_hbm, word_hbm, ttt_hbm, pos_hbm, out_hbm,
          comb_v, ttrow_v, idx_v, tts_v, rows_v, sp_v,
          gs0, gs1, gs2, gs3, xs0, xs1, xs2, xs3, ws0, ws1, isem):
    gsem = (gs0, gs1, gs2, gs3)
    xsem = (xs0, xs1, xs2, xs3)
    wid = lax.axis_index("s") * NC + lax.axis_index("c")
    wbase = wid * TOK_PER_W

    idx_copy = pltpu.make_async_copy(
        ids_hbm.at[pl.ds(wbase, TOK_PER_W)], idx_v, isem)
    idx_copy.start()
    tts_copy = pltpu.make_async_copy(
        tt_hbm.at[pl.ds(wbase, TOK_PER_W)], tts_v, isem)
    tts_copy.start()

    pltpu.sync_copy(pos_hbm.at[pl.ds(0, S)], comb_v.at[pl.ds(0, S)])
    pltpu.sync_copy(pos_hbm.at[pl.ds(0, S)], comb_v.at[pl.ds(S, S)])
    pltpu.sync_copy(ttt_hbm, ttrow_v)

    def build(s, _):
        for j in range(NSL):
            sl = pl.ds(16 * j, 16)
            comb_v[s, sl] += ttrow_v[0, sl]
            comb_v[S + s, sl] += ttrow_v[1, sl]
        return 0

    lax.fori_loop(0, S, build, 0)
    idx_copy.wait()
    tts_copy.wait()

    def start_gather(c, b):
        pltpu.make_async_copy(
            word_hbm.at[idx_v.at[pl.ds(c * CHUNK, CHUNK)]],
            rows_v.at[b], gsem[b]).start()

    def wait_gather(b):
        pltpu.make_async_copy(
            word_hbm.at[idx_v.at[pl.ds(0, CHUNK)]],
            rows_v.at[b], gsem[b]).wait()

    sid = lax.axis_index("s")
    wsp = (ws0, ws1)

    def start_xbar(b, p):
        pltpu.make_async_copy(
            rows_v.at[b], sp_v.at[sid, p], xsem[b]).start()

    def wait_xbar(b):
        pltpu.make_async_copy(
            rows_v.at[b], sp_v.at[sid, 0], xsem[b]).wait()

    def start_spwrite(c, p):
        pltpu.make_async_copy(
            sp_v.at[sid, p], out_hbm.at[pl.ds(wbase + c * CHUNK, CHUNK)],
            wsp[p]).start()

    def wait_spwrite(p):
        pltpu.make_async_copy(
            sp_v.at[sid, 0], out_hbm.at[pl.ds(wbase, CHUNK)], wsp[p]).wait()

    start_gather(0, 0)
    start_gather(1, 1)

    def outer(i, _):
        for b in range(NBUF):
            c = i * NBUF + b
            wait_gather(b)

            def tok_group(g, _):
                g16 = g * 16
                tv = tts_v[pl.ds(c * CHUNK + g16, 16)]

                def keyof(t):
                    p = lax.rem(c * CHUNK + g16 + t, S)
                    return tv[t] * S + p

                k0 = keyof(0)
                cs = [comb_v[k0, pl.ds(16 * j, 16)] for j in range(NSL)]
                rs = [rows_v[b, g16, pl.ds(16 * j, 16)] for j in range(NSL)]
                for t in range(16):
                    s = g16 + t
                    has_next = t + 1 < 16
                    if has_next:
                        k1 = keyof(t + 1)
                        nrs = [rows_v[b, s + 1, pl.ds(16 * j, 16)]
                               for j in range(NSL)]
                        ncs = []
                    for j in range(NSL):
                        if has_next:
                            ncs.append(comb_v[k1, pl.ds(16 * j, 16)])
                        rows_v[b, s, pl.ds(16 * j, 16)] = rs[j] + cs[j]
                    if has_next:
                        cs, rs = ncs, nrs
                return 0

            lax.fori_loop(0, CHUNK // 16, tok_group, 0)

            bp = (b - 1) % NBUF
            pp = (b - 1) % 2

            def drain_prev():
                wait_xbar(bp)
                start_spwrite(c - 1, pp)

            if b == 0:
                @pl.when(c >= 1)
                def _():
                    drain_prev()
            else:
                drain_prev()

            p = b % 2

            def drain_slot():
                wait_spwrite(p)

            if b < 2:
                @pl.when(c >= 2)
                def _():
                    drain_slot()
            else:
                drain_slot()

            start_xbar(b, p)

            cn = c + 2
            bn = (b + 2) % NBUF

            @pl.when(cn < N_CHUNKS)
            def _():
                start_gather(cn, bn)
        return 0

    lax.fori_loop(0, N_OUTER, outer, 0)
    wait_xbar((N_CHUNKS - 1) % NBUF)
    start_spwrite(N_CHUNKS - 1, (N_CHUNKS - 1) % 2)
    wait_spwrite((N_CHUNKS - 2) % 2)
    wait_spwrite((N_CHUNKS - 1) % 2)


@jax.jit
def _embed(ids_flat, tt_flat, word_table, token_type_table, position_table):
    mesh = plsc.VectorSubcoreMesh(core_axis_name="c", subcore_axis_name="s")
    k = functools.partial(
        pl.kernel,
        mesh=mesh,
        out_type=jax.ShapeDtypeStruct((N_TOK, HIDDEN), jnp.float32),
        scratch_types=[
            pltpu.VMEM((2 * S, HIDDEN), jnp.float32),
            pltpu.VMEM((2, HIDDEN), jnp.float32),
            pltpu.VMEM((TOK_PER_W,), jnp.int32),
            pltpu.VMEM((TOK_PER_W,), jnp.int32),
            pltpu.VMEM((NBUF, CHUNK, HIDDEN), jnp.float32),
            pltpu.VMEM_SHARED((NS, 2, CHUNK, HIDDEN), jnp.float32),
        ] + [pltpu.SemaphoreType.DMA] * (NBUF + 4 + 2 + 1),
    )(_body)
    return k(ids_flat, tt_flat, word_table, token_type_table, position_table)


def kernel(inputs_ids, token_type_ids, word_table, token_type_table, position_table):
    ids_flat = inputs_ids.reshape(-1).astype(jnp.int32)
    tt_flat = token_type_ids.reshape(-1).astype(jnp.int32)
    out = _embed(ids_flat, tt_flat, word_table, token_type_table, position_table)
    wt_out = word_table + jnp.float32(0.0)
    return (out.reshape(inputs_ids.shape[0], inputs_ids.shape[1], HIDDEN), wt_out)


# speedup vs baseline: 1.0147x; 1.0147x over previous
import functools

import jax
import jax.numpy as jnp
from jax import lax
from jax.experimental import pallas as pl
from jax.experimental.pallas import tpu as pltpu
from jax.experimental.pallas import tpu_sc as plsc

VOCAB = 100000
HIDDEN = 128
S = 200
B = 1024

NC = 2
NS = 16
NW = NC * NS
N_TOK = B * S
TOK_PER_W = N_TOK // NW
CHUNK = 80
N_CHUNKS = TOK_PER_W // CHUNK
NBUF = 4
N_OUTER = N_CHUNKS // NBUF
NSL = HIDDEN // 16


def _body(ids_hbm, tt_hbm, word_hbm, ttt_hbm, pos_hbm, out_hbm,
          comb_v, ttrow_v, idx_v, tts_v, rows_v, sp_v,
          gs0, gs1, gs2, gs3, xs0, xs1, xs2, xs3, ws0, ws1, isem):
    gsem = (gs0, gs1, gs2, gs3)
    xsem = (xs0, xs1, xs2, xs3)
    wid = lax.axis_index("s") * NC + lax.axis_index("c")
    wbase = wid * TOK_PER_W

    idx_copy = pltpu.make_async_copy(
        ids_hbm.at[pl.ds(wbase, TOK_PER_W)], idx_v, isem)
    idx_copy.start()
    tts_copy = pltpu.make_async_copy(
        tt_hbm.at[pl.ds(wbase, TOK_PER_W)], tts_v, isem)
    tts_copy.start()

    pltpu.sync_copy(pos_hbm.at[pl.ds(0, S)], comb_v.at[pl.ds(0, S)])
    pltpu.sync_copy(pos_hbm.at[pl.ds(0, S)], comb_v.at[pl.ds(S, S)])
    pltpu.sync_copy(ttt_hbm, ttrow_v)

    def build(s, _):
        for j in range(NSL):
            sl = pl.ds(16 * j, 16)
            comb_v[s, sl] += ttrow_v[0, sl]
            comb_v[S + s, sl] += ttrow_v[1, sl]
        return 0

    lax.fori_loop(0, S, build, 0)
    idx_copy.wait()
    tts_copy.wait()

    def start_gather(c, b):
        pltpu.make_async_copy(
            word_hbm.at[idx_v.at[pl.ds(c * CHUNK, CHUNK)]],
            rows_v.at[b], gsem[b]).start()

    def wait_gather(b):
        pltpu.make_async_copy(
            word_hbm.at[idx_v.at[pl.ds(0, CHUNK)]],
            rows_v.at[b], gsem[b]).wait()

    sid = lax.axis_index("s")
    wsp = (ws0, ws1)

    def start_xbar(b, p):
        pltpu.make_async_copy(
            rows_v.at[b], sp_v.at[sid, p], xsem[b]).start()

    def wait_xbar(b):
        pltpu.make_async_copy(
            rows_v.at[b], sp_v.at[sid, 0], xsem[b]).wait()

    def start_spwrite(c, p):
        pltpu.make_async_copy(
            sp_v.at[sid, p], out_hbm.at[pl.ds(wbase + c * CHUNK, CHUNK)],
            wsp[p]).start()

    def wait_spwrite(p):
        pltpu.make_async_copy(
            sp_v.at[sid, 0], out_hbm.at[pl.ds(wbase, CHUNK)], wsp[p]).wait()

    start_gather(0, 0)
    start_gather(1, 1)

    def outer(i, _):
        for b in range(NBUF):
            c = i * NBUF + b
            wait_gather(b)

            def tok_group(g, _):
                g16 = g * 16
                tv = tts_v[pl.ds(c * CHUNK + g16, 16)]

                def keyof(t):
                    p = lax.rem(c * CHUNK + g16 + t, S)
                    return tv[t] * S + p

                k0 = keyof(0)
                cs = [comb_v[k0, pl.ds(16 * j, 16)] for j in range(NSL)]
                rs = [rows_v[b, g16, pl.ds(16 * j, 16)] for j in range(NSL)]
                for t in range(16):
                    s = g16 + t
                    has_next = t + 1 < 16
                    if has_next:
                        k1 = keyof(t + 1)
                        nrs = [rows_v[b, s + 1, pl.ds(16 * j, 16)]
                               for j in range(NSL)]
                        ncs = []
                    for j in range(NSL):
                        if has_next:
                            ncs.append(comb_v[k1, pl.ds(16 * j, 16)])
                        rows_v[b, s, pl.ds(16 * j, 16)] = rs[j] + cs[j]
                    if has_next:
                        cs, rs = ncs, nrs
                return 0

            lax.fori_loop(0, CHUNK // 16, tok_group, 0)

            bp = (b - 1) % NBUF
            pp = (b - 1) % 2

            def drain_prev():
                wait_xbar(bp)
                start_spwrite(c - 1, pp)

            if b == 0:
                @pl.when(c >= 1)
                def _():
                    drain_prev()
            else:
                drain_prev()

            p = b % 2

            def drain_slot():
                wait_spwrite(p)

            if b < 2:
                @pl.when(c >= 2)
                def _():
                    drain_slot()
            else:
                drain_slot()

            start_xbar(b, p)

            cn = c + 2
            bn = (b + 2) % NBUF

            @pl.when(cn < N_CHUNKS)
            def _():
                start_gather(cn, bn)
        return 0

    lax.fori_loop(0, N_OUTER, outer, 0)
    wait_xbar((N_CHUNKS - 1) % NBUF)
    start_spwrite(N_CHUNKS - 1, (N_CHUNKS - 1) % 2)
    wait_spwrite((N_CHUNKS - 2) % 2)
    wait_spwrite((N_CHUNKS - 1) % 2)


@jax.jit
def _embed(ids_flat, tt_flat, word_table, token_type_table, position_table):
    mesh = plsc.VectorSubcoreMesh(core_axis_name="c", subcore_axis_name="s")
    k = functools.partial(
        pl.kernel,
        mesh=mesh,
        out_type=jax.ShapeDtypeStruct((N_TOK, HIDDEN), jnp.float32),
        scratch_types=[
            pltpu.VMEM((2 * S, HIDDEN), jnp.float32),
            pltpu.VMEM((2, HIDDEN), jnp.float32),
            pltpu.VMEM((TOK_PER_W,), jnp.int32),
            pltpu.VMEM((TOK_PER_W,), jnp.int32),
            pltpu.VMEM((NBUF, CHUNK, HIDDEN), jnp.float32),
            pltpu.VMEM_SHARED((NS, 2, CHUNK, HIDDEN), jnp.float32),
        ] + [pltpu.SemaphoreType.DMA] * (NBUF + 4 + 2 + 1),
    )(_body)
    return k(ids_flat, tt_flat, word_table, token_type_table, position_table)


def kernel(inputs_ids, token_type_ids, word_table, token_type_table, position_table):
    ids_flat = inputs_ids.reshape(-1).astype(jnp.int32)
    tt_flat = token_type_ids.reshape(-1).astype(jnp.int32)
    wt_out = word_table + jnp.float32(0.0)
    out = _embed(ids_flat, tt_flat, word_table, token_type_table, position_table)
    return (out.reshape(inputs_ids.shape[0], inputs_ids.shape[1], HIDDEN), wt_out)
